# baseline (speedup 1.0000x reference)
import functools

import jax
import jax.numpy as jnp
from jax import lax
from jax.experimental import pallas as pl
from jax.experimental.pallas import tpu as pltpu
from jax.experimental.pallas import tpu_sc as plsc

_SELU_ALPHA = 1.6732632423543772
_SELU_SCALE = 1.0507009873554805
_PACK = 16


def _selu(v):
    return _SELU_SCALE * jnp.where(v > 0, v, _SELU_ALPHA * (jnp.exp(v) - 1.0))


def _sigmoid(v):
    return 1.0 / (1.0 + jnp.exp(-v))


def _round_up(a, b):
    return -(-a // b) * b


def _pick_block(total, cap, mult):
    for k in range(cap - cap % mult, mult - 1, -mult):
        if total % k == 0:
            return k
    return None


def _expand(w):
    return jnp.kron(jnp.eye(_PACK, dtype=jnp.float32), w)


def _expand_bias(b):
    return jnp.tile(b, _PACK).reshape(1, -1)


def _lin_selu_body(x_ref, w_ref, b_ref, o_ref):
    o_ref[...] = _selu(
        jnp.dot(x_ref[...], w_ref[...], preferred_element_type=jnp.float32)
        + b_ref[...])


def _tc_lin_selu(xk, w_e, b_e):
    return pl.pallas_call(
        _lin_selu_body,
        out_shape=jax.ShapeDtypeStruct(xk.shape, jnp.float32),
    )(xk, w_e, b_e)


def _gru_body(aggp_ref, degp_ref, h_ref,
              wir_ref, wiz_ref, win_ref, bi_ref,
              whr_ref, whz_ref, whn_ref, bh_ref,
              w2_ref, b2_ref, o_ref):
    deg = degp_ref[0] + degp_ref[1]
    rdeg = 1.0 / jnp.maximum(deg, 1.0)
    agg = (aggp_ref[0] + aggp_ref[1]) * rdeg
    h = h_ref[...]

    def mm(a, wref):
        return jnp.dot(a, wref[...], preferred_element_type=jnp.float32)

    bi = bi_ref[...]
    bh = bh_ref[...]
    r = _sigmoid(mm(agg, wir_ref) + bi[:, 0:128]
                 + mm(h, whr_ref) + bh[:, 0:128])
    z = _sigmoid(mm(agg, wiz_ref) + bi[:, 128:256]
                 + mm(h, whz_ref) + bh[:, 128:256])
    nn = jnp.tanh(mm(agg, win_ref) + bi[:, 256:384]
                  + r * (mm(h, whn_ref) + bh[:, 256:384]))
    xn = (1.0 - z) * nn + z * h
    o_ref[...] = _selu(mm(xn, w2_ref) + b2_ref[...])


def _tc_gru(aggk, degk, hk, wir, wiz, win, bi3, whr, whz, whn, bh3, w2e, b2e):
    return pl.pallas_call(
        _gru_body,
        out_shape=jax.ShapeDtypeStruct(hk.shape, jnp.float32),
    )(aggk, degk, hk, wir, wiz, win, bi3, whr, whz, whn, bh3, w2e, b2e)


def _head_body(sums_ref, cnt_ref, w2_ref, b2_ref, w3_ref, b3_ref,
               xo_ref, xf_ref):
    sums = sums_ref[0] + sums_ref[1]
    cnt = cnt_ref[0] + cnt_ref[1]
    x5 = sums / jnp.maximum(cnt, 1.0)
    nrm = jnp.sqrt(jnp.sum(x5 * x5, axis=-1, keepdims=True))
    x7 = x5 / jnp.maximum(nrm, 1e-12)
    xf = _selu(jnp.dot(x7, w2_ref[...], preferred_element_type=jnp.float32)
               + b2_ref[...])
    xf_ref[...] = xf
    xo_ref[...] = (jnp.dot(xf, w3_ref[...], preferred_element_type=jnp.float32)
                   + b3_ref[...])


def _tc_head(sumsp, cntp, wfc2, bfc2, wfc3, bfc3):
    nc, g, hd = sumsp.shape
    f2 = wfc2.shape[1]
    return pl.pallas_call(
        _head_body,
        out_shape=(jax.ShapeDtypeStruct((g, 1), jnp.float32),
                   jax.ShapeDtypeStruct((g, f2), jnp.float32)),
    )(sumsp, cntp, wfc2, bfc2.reshape(1, f2),
      wfc3, bfc3.reshape(1, 1))


def _sc_edge_pass(table, icat, zeros8, k, nchunks, nc, ns):
    npad, hd = table.shape
    rows_per_tile = npad // ns
    npairs = nchunks // 2
    mesh = plsc.VectorSubcoreMesh(core_axis_name="c", subcore_axis_name="s")

    @functools.partial(
        pl.kernel,
        out_type=[jax.ShapeDtypeStruct((nc, npad, hd), jnp.float32)],
        mesh=mesh,
        scratch_types=[
            pltpu.VMEM((2, k), jnp.int32), pltpu.VMEM((2, k), jnp.int32),
            pltpu.VMEM((k, hd), jnp.float32), pltpu.VMEM((k, hd), jnp.float32),
            pltpu.VMEM_SHARED((npad, hd), jnp.float32),
            pltpu.SemaphoreType.DMA, pltpu.SemaphoreType.DMA,
        ],
        compiler_params=pltpu.CompilerParams(use_tc_tiling_on_sc=False))
    def run(table_h, icat_h, z8_h, agg_o,
            idx0, idx1, rows0, rows1, agg_sh, gs0, gs1):
        cid = lax.axis_index("c")
        sid = lax.axis_index("s")
        wid = sid * nc + cid

        @pl.when(sid == 0)
        def _zero():
            pltpu.sync_copy(z8_h, agg_sh)
        plsc.subcore_barrier()

        base_c = wid * nchunks

        pltpu.sync_copy(icat_h.at[base_c], idx0)
        pltpu.async_copy(table_h.at[idx0.at[0]], rows0, gs0)

        def pair(j, carry):
            pltpu.sync_copy(icat_h.at[base_c + 2 * j + 1], idx1)
            pltpu.async_copy(table_h.at[idx1.at[0]], rows1, gs1)
            pltpu.make_async_copy(table_h.at[idx0.at[0]], rows0, gs0).wait()
            pltpu.sync_copy(rows0, agg_sh.at[idx0.at[1]], add=True)

            @pl.when(j < npairs - 1)
            def _pref():
                pltpu.sync_copy(icat_h.at[base_c + 2 * j + 2], idx0)
                pltpu.async_copy(table_h.at[idx0.at[0]], rows0, gs0)

            pltpu.make_async_copy(table_h.at[idx1.at[0]], rows1, gs1).wait()
            pltpu.sync_copy(rows1, agg_sh.at[idx1.at[1]], add=True)
            return carry

        lax.fori_loop(0, npairs, pair, 0)
        plsc.subcore_barrier()

        off = sid * rows_per_tile
        pltpu.sync_copy(agg_sh.at[pl.ds(off, rows_per_tile)],
                        agg_o.at[cid, pl.ds(off, rows_per_tile)])

    return run(table, icat, zeros8)[0]


def _sc_deg(icat, ones_k, zeros8, k, nchunks, nc, ns, npad):
    hd = ones_k.shape[1]
    rows_per_tile = npad // ns
    mesh = plsc.VectorSubcoreMesh(core_axis_name="c", subcore_axis_name="s")

    @functools.partial(
        pl.kernel,
        out_type=[jax.ShapeDtypeStruct((nc, npad, hd), jnp.float32)],
        mesh=mesh,
        scratch_types=[
            pltpu.VMEM((k,), jnp.int32), pltpu.VMEM((k,), jnp.int32),
            pltpu.VMEM((k, hd), jnp.float32),
            pltpu.VMEM_SHARED((npad, hd), jnp.float32),
            pltpu.SemaphoreType.DMA,
        ],
        compiler_params=pltpu.CompilerParams(use_tc_tiling_on_sc=False))
    def run(icat_h, ones_h, z8_h, deg_o, dst0, dst1, ones_v, deg_sh, isem):
        cid = lax.axis_index("c")
        sid = lax.axis_index("s")
        wid = sid * nc + cid

        @pl.when(sid == 0)
        def _zero():
            pltpu.sync_copy(z8_h, deg_sh)
        pltpu.sync_copy(ones_h, ones_v)
        plsc.subcore_barrier()

        base_c = wid * nchunks
        npairs = nchunks // 2
        pltpu.sync_copy(icat_h.at[base_c, 1], dst0)

        def pair(j, carry):
            c1 = base_c + 2 * j + 1
            pltpu.async_copy(icat_h.at[c1, 1], dst1, isem)
            pltpu.sync_copy(ones_v, deg_sh.at[dst0], add=True)
            pltpu.make_async_copy(icat_h.at[c1, 1], dst1, isem).wait()

            @pl.when(j < npairs - 1)
            def _pref():
                c2 = base_c + 2 * j + 2
                pltpu.async_copy(icat_h.at[c2, 1], dst0, isem)

            pltpu.sync_copy(ones_v, deg_sh.at[dst1], add=True)

            @pl.when(j < npairs - 1)
            def _drain():
                c2 = base_c + 2 * j + 2
                pltpu.make_async_copy(icat_h.at[c2, 1], dst0, isem).wait()
            return carry

        lax.fori_loop(0, npairs, pair, 0)
        plsc.subcore_barrier()

        off = sid * rows_per_tile
        pltpu.sync_copy(deg_sh.at[pl.ds(off, rows_per_tile)],
                        deg_o.at[cid, pl.ds(off, rows_per_tile)])

    return run(icat, ones_k, zeros8)[0]


def _sc_pool(rows, batch, ones_k, zeros8, k, nc, ns):
    npool, hd = rows.shape
    gp = zeros8.shape[0]
    mesh = plsc.VectorSubcoreMesh(core_axis_name="c", subcore_axis_name="s")

    @functools.partial(
        pl.kernel,
        out_type=[jax.ShapeDtypeStruct((nc, gp, hd), jnp.float32),
                  jax.ShapeDtypeStruct((nc, gp, hd), jnp.float32)],
        mesh=mesh,
        scratch_types=[
            pltpu.VMEM((k,), jnp.int32),
            pltpu.VMEM((k, hd), jnp.float32),
            pltpu.VMEM((k, hd), jnp.float32),
            pltpu.VMEM_SHARED((gp, hd), jnp.float32),
            pltpu.VMEM_SHARED((gp, hd), jnp.float32),
        ],
        compiler_params=pltpu.CompilerParams(use_tc_tiling_on_sc=False))
    def run(rows_h, batch_h, ones_h, z8_h, sums_o, cnt_o,
            dst_v, rows_v, ones_v, sums_sh, cnt_sh):
        cid = lax.axis_index("c")
        sid = lax.axis_index("s")
        wid = sid * nc + cid

        @pl.when(sid == 0)
        def _zero():
            pltpu.sync_copy(z8_h, sums_sh)
            pltpu.sync_copy(z8_h, cnt_sh)

        pltpu.sync_copy(ones_h, ones_v)
        plsc.subcore_barrier()

        base = wid * k
        pltpu.sync_copy(batch_h.at[pl.ds(base, k)], dst_v)
        pltpu.sync_copy(rows_h.at[pl.ds(base, k)], rows_v)
        pltpu.sync_copy(rows_v, sums_sh.at[dst_v], add=True)
        pltpu.sync_copy(ones_v, cnt_sh.at[dst_v], add=True)
        plsc.subcore_barrier()

        @pl.when(sid == 0)
        def _out():
            pltpu.sync_copy(sums_sh, sums_o.at[cid])
            pltpu.sync_copy(cnt_sh, cnt_o.at[cid])

    return run(rows, batch, ones_k, zeros8)


def kernel(x, x_ex, DFS, STATUS, edge_index, batch,
           W1, b1, Wi1, bi1, Wh1, bh1, Wi2, bi2, Wh2, bh2,
           W2, b2, Wfc2, bfc2, Wfc3, bfc3):
    n, f_in = x.shape
    e = edge_index.shape[1]
    g = DFS.shape[0]
    hd = W1.shape[1]

    info = plsc.get_sparse_core_info()
    nc, ns = info.num_cores, info.num_subcores
    nw = nc * ns

    k = None
    if e % (2 * nw) == 0:
        k = _pick_block(e // (2 * nw), 4096, 16)
    if k is None:
        k = 2048
        epad = _round_up(e, 2 * nw * k)
    else:
        epad = e
    nchunks = epad // (nw * k)
    pad_rows = 128 if epad > e else 0
    npad = _round_up(n + pad_rows, _round_up(nw * 16, 128))
    np16 = npad * hd // 128

    src_e, dst_e = edge_index[0], edge_index[1]
    if epad > e:
        npe = epad - e
        pad_idx = (n + (jnp.arange(npe, dtype=jnp.int32) % pad_rows))
        src_e = jnp.concatenate([src_e, pad_idx])
        dst_e = jnp.concatenate([dst_e, pad_idx])
    nct = epad // k
    icat = jnp.stack([src_e.reshape(nct, k), dst_e.reshape(nct, k)], axis=1)

    ones_k = jnp.ones((k, hd), jnp.float32)
    zeros8 = jnp.zeros((npad, hd), jnp.float32)

    kp = npad // nw
    npool = npad
    gp = _round_up(g + 16, 16)
    batch_p = jnp.concatenate(
        [batch, jnp.full((npool - n,), g, jnp.int32)])
    ones_kp = jnp.ones((kp, hd), jnp.float32)
    zeros_g8 = jnp.zeros((gp, hd), jnp.float32)

    w1e, b1e = _expand(W1), _expand_bias(b1)
    wir1, wiz1, win1 = (_expand(Wi1[:, 0:hd]), _expand(Wi1[:, hd:2 * hd]),
                        _expand(Wi1[:, 2 * hd:3 * hd]))
    whr1, whz1, whn1 = (_expand(Wh1[:, 0:hd]), _expand(Wh1[:, hd:2 * hd]),
                        _expand(Wh1[:, 2 * hd:3 * hd]))
    bi1e = jnp.concatenate([_expand_bias(bi1[0:hd]),
                            _expand_bias(bi1[hd:2 * hd]),
                            _expand_bias(bi1[2 * hd:3 * hd])], axis=1)
    bh1e = jnp.concatenate([_expand_bias(bh1[0:hd]),
                            _expand_bias(bh1[hd:2 * hd]),
                            _expand_bias(bh1[2 * hd:3 * hd])], axis=1)
    wir2, wiz2, win2 = (_expand(Wi2[:, 0:hd]), _expand(Wi2[:, hd:2 * hd]),
                        _expand(Wi2[:, 2 * hd:3 * hd]))
    whr2, whz2, whn2 = (_expand(Wh2[:, 0:hd]), _expand(Wh2[:, hd:2 * hd]),
                        _expand(Wh2[:, 2 * hd:3 * hd]))
    bi2e = jnp.concatenate([_expand_bias(bi2[0:hd]),
                            _expand_bias(bi2[hd:2 * hd]),
                            _expand_bias(bi2[2 * hd:3 * hd])], axis=1)
    bh2e = jnp.concatenate([_expand_bias(bh2[0:hd]),
                            _expand_bias(bh2[hd:2 * hd]),
                            _expand_bias(bh2[2 * hd:3 * hd])], axis=1)
    w2e, b2e = _expand(W2), _expand_bias(b2)

    xp = jnp.concatenate([x, jnp.zeros((npad - n, f_in), jnp.float32)])
    xk = xp.reshape(np16, 128)

    x1k = _tc_lin_selu(xk, w1e, b1e)
    table1 = x1k.reshape(npad, hd)

    degp = _sc_deg(icat, ones_k, zeros8, k, nchunks, nc, ns, npad)
    degk = degp.reshape(nc, np16, 128)

    agg1p = _sc_edge_pass(table1, icat, zeros8, k, nchunks, nc, ns)
    agg1k = agg1p.reshape(nc, np16, 128)

    x2k = _tc_gru(agg1k, degk, x1k, wir1, wiz1, win1, bi1e,
                  whr1, whz1, whn1, bh1e, w2e, b2e)
    table2 = x2k.reshape(npad, hd)

    agg2p = _sc_edge_pass(table2, icat, zeros8, k, nchunks, nc, ns)
    agg2k = agg2p.reshape(nc, np16, 128)

    x4k = _tc_gru(agg2k, degk, x2k, wir2, wiz2, win2, bi2e,
                  whr2, whz2, whn2, bh2e, w2e, b2e)
    x4s = x4k.reshape(npad, hd)

    sumsp, cntp = _sc_pool(x4s, batch_p, ones_kp, zeros_g8, kp, nc, ns)

    x_out, x_feat = _tc_head(sumsp[:, :g], cntp[:, :g, 0:1], Wfc2, bfc2,
                             Wfc3, bfc3)
    return (x_out, x_feat)


# speedup vs baseline: 1.9763x; 1.9763x over previous
import functools

import jax
import jax.numpy as jnp
from jax import lax
from jax.experimental import pallas as pl
from jax.experimental.pallas import tpu as pltpu
from jax.experimental.pallas import tpu_sc as plsc

_SELU_ALPHA = 1.6732632423543772
_SELU_SCALE = 1.0507009873554805
_PACK = 16


def _selu(v):
    return _SELU_SCALE * jnp.where(v > 0, v, _SELU_ALPHA * (jnp.exp(v) - 1.0))


def _sigmoid(v):
    return 1.0 / (1.0 + jnp.exp(-v))


def _round_up(a, b):
    return -(-a // b) * b


def _pick_block(total, cap, mult):
    for k in range(cap - cap % mult, mult - 1, -mult):
        if total % k == 0:
            return k
    return None


def _expand(w):
    return jnp.kron(jnp.eye(_PACK, dtype=jnp.float32), w)


def _expand_bias(b):
    return jnp.tile(b, _PACK).reshape(1, -1)


def _lin_selu_body(x_ref, w_ref, b_ref, o_ref):
    o_ref[...] = _selu(
        jnp.dot(x_ref[...], w_ref[...], preferred_element_type=jnp.float32)
        + b_ref[...])


def _tc_lin_selu(xk, w_e, b_e):
    return pl.pallas_call(
        _lin_selu_body,
        out_shape=jax.ShapeDtypeStruct(xk.shape, jnp.float32),
    )(xk, w_e, b_e)


def _gru_body(aggp_ref, degp_ref, h_ref,
              wir_ref, wiz_ref, win_ref, bi_ref,
              whr_ref, whz_ref, whn_ref, bh_ref,
              w2_ref, b2_ref, o_ref):
    deg = degp_ref[0] + degp_ref[1]
    rdeg = 1.0 / jnp.maximum(deg, 1.0)
    agg = (aggp_ref[0] + aggp_ref[1]) * rdeg
    h = h_ref[...]

    def mm(a, wref):
        return jnp.dot(a, wref[...], preferred_element_type=jnp.float32)

    bi = bi_ref[...]
    bh = bh_ref[...]
    r = _sigmoid(mm(agg, wir_ref) + bi[:, 0:128]
                 + mm(h, whr_ref) + bh[:, 0:128])
    z = _sigmoid(mm(agg, wiz_ref) + bi[:, 128:256]
                 + mm(h, whz_ref) + bh[:, 128:256])
    nn = jnp.tanh(mm(agg, win_ref) + bi[:, 256:384]
                  + r * (mm(h, whn_ref) + bh[:, 256:384]))
    xn = (1.0 - z) * nn + z * h
    o_ref[...] = _selu(mm(xn, w2_ref) + b2_ref[...])


def _tc_gru(aggk, degk, hk, wir, wiz, win, bi3, whr, whz, whn, bh3, w2e, b2e):
    return pl.pallas_call(
        _gru_body,
        out_shape=jax.ShapeDtypeStruct(hk.shape, jnp.float32),
    )(aggk, degk, hk, wir, wiz, win, bi3, whr, whz, whn, bh3, w2e, b2e)


def _head_body(sums_ref, cnt_ref, w2_ref, b2_ref, w3_ref, b3_ref,
               xo_ref, xf_ref):
    sums = sums_ref[0] + sums_ref[1]
    cnt = cnt_ref[0] + cnt_ref[1]
    x5 = sums / jnp.maximum(cnt, 1.0)
    nrm = jnp.sqrt(jnp.sum(x5 * x5, axis=-1, keepdims=True))
    x7 = x5 / jnp.maximum(nrm, 1e-12)
    xf = _selu(jnp.dot(x7, w2_ref[...], preferred_element_type=jnp.float32)
               + b2_ref[...])
    xf_ref[...] = xf
    xo_ref[...] = (jnp.dot(xf, w3_ref[...], preferred_element_type=jnp.float32)
                   + b3_ref[...])


def _tc_head(sumsp, cntp, wfc2, bfc2, wfc3, bfc3):
    nc, g, hd = sumsp.shape
    f2 = wfc2.shape[1]
    return pl.pallas_call(
        _head_body,
        out_shape=(jax.ShapeDtypeStruct((g, 1), jnp.float32),
                   jax.ShapeDtypeStruct((g, f2), jnp.float32)),
    )(sumsp, cntp, wfc2, bfc2.reshape(1, f2),
      wfc3, bfc3.reshape(1, 1))


def _sc_edge_pass(table, src_e, dst_e, zeros8, k, nchunks, nc, ns):
    npad, hd = table.shape
    per_w = k * nchunks
    rows_per_tile = npad // ns
    nquads = nchunks // 4
    mesh = plsc.VectorSubcoreMesh(core_axis_name="c", subcore_axis_name="s")

    @functools.partial(
        pl.kernel,
        out_type=[jax.ShapeDtypeStruct((nc, npad, hd), jnp.float32)],
        mesh=mesh,
        scratch_types=[
            [pltpu.VMEM((k,), jnp.int32)] * 4,
            [pltpu.VMEM((k,), jnp.int32)] * 4,
            [pltpu.VMEM((k, hd), jnp.float32)] * 4,
            pltpu.VMEM_SHARED((npad, hd), jnp.float32),
            [pltpu.SemaphoreType.DMA] * 4,
        ],
        compiler_params=pltpu.CompilerParams(use_tc_tiling_on_sc=False))
    def run(table_h, srce_h, dste_h, z8_h, agg_o,
            srcs, dsts, rowss, agg_sh, gsems):
        cid = lax.axis_index("c")
        sid = lax.axis_index("s")
        wid = sid * nc + cid

        @pl.when(sid == 0)
        def _zero():
            pltpu.sync_copy(z8_h, agg_sh)
        plsc.subcore_barrier()

        base_w = wid * per_w

        for b in range(4):
            pltpu.sync_copy(srce_h.at[pl.ds(base_w + b * k, k)], srcs[b])
            pltpu.sync_copy(dste_h.at[pl.ds(base_w + b * k, k)], dsts[b])
            pltpu.async_copy(table_h.at[srcs[b]], rowss[b], gsems[b])

        def quad(j, carry):
            for b in range(4):
                pltpu.make_async_copy(table_h.at[srcs[b]], rowss[b],
                                      gsems[b]).wait()
                pltpu.sync_copy(rowss[b], agg_sh.at[dsts[b]], add=True)

                @pl.when(j < nquads - 1)
                def _pref(b=b):
                    nb = base_w + (4 * j + b + 4) * k
                    pltpu.sync_copy(srce_h.at[pl.ds(nb, k)], srcs[b])
                    pltpu.sync_copy(dste_h.at[pl.ds(nb, k)], dsts[b])
                    pltpu.async_copy(table_h.at[srcs[b]], rowss[b], gsems[b])
            return carry

        lax.fori_loop(0, nquads, quad, 0)
        plsc.subcore_barrier()

        off = sid * rows_per_tile
        pltpu.sync_copy(agg_sh.at[pl.ds(off, rows_per_tile)],
                        agg_o.at[cid, pl.ds(off, rows_per_tile)])

    return run(table, src_e, dst_e, zeros8)[0]


def _sc_deg(dst_e, ones_k, zeros8, k, nchunks, nc, ns, npad):
    hd = ones_k.shape[1]
    per_w = k * nchunks
    rows_per_tile = npad // ns
    mesh = plsc.VectorSubcoreMesh(core_axis_name="c", subcore_axis_name="s")

    @functools.partial(
        pl.kernel,
        out_type=[jax.ShapeDtypeStruct((nc, npad, hd), jnp.float32)],
        mesh=mesh,
        scratch_types=[
            pltpu.VMEM((k,), jnp.int32), pltpu.VMEM((k,), jnp.int32),
            pltpu.VMEM((k, hd), jnp.float32),
            pltpu.VMEM_SHARED((npad, hd), jnp.float32),
            pltpu.SemaphoreType.DMA,
        ],
        compiler_params=pltpu.CompilerParams(use_tc_tiling_on_sc=False))
    def run(dste_h, ones_h, z8_h, deg_o, dst0, dst1, ones_v, deg_sh, isem):
        cid = lax.axis_index("c")
        sid = lax.axis_index("s")
        wid = sid * nc + cid

        @pl.when(sid == 0)
        def _zero():
            pltpu.sync_copy(z8_h, deg_sh)
        pltpu.sync_copy(ones_h, ones_v)
        plsc.subcore_barrier()

        base_w = wid * per_w
        npairs = nchunks // 2
        pltpu.sync_copy(dste_h.at[pl.ds(base_w, k)], dst0)

        def pair(j, carry):
            b1 = base_w + (2 * j + 1) * k
            pltpu.async_copy(dste_h.at[pl.ds(b1, k)], dst1, isem)
            pltpu.sync_copy(ones_v, deg_sh.at[dst0], add=True)
            pltpu.make_async_copy(dste_h.at[pl.ds(b1, k)], dst1, isem).wait()

            @pl.when(j < npairs - 1)
            def _pref():
                b2 = base_w + (2 * j + 2) * k
                pltpu.async_copy(dste_h.at[pl.ds(b2, k)], dst0, isem)

            pltpu.sync_copy(ones_v, deg_sh.at[dst1], add=True)

            @pl.when(j < npairs - 1)
            def _drain():
                b2 = base_w + (2 * j + 2) * k
                pltpu.make_async_copy(dste_h.at[pl.ds(b2, k)], dst0,
                                      isem).wait()
            return carry

        lax.fori_loop(0, npairs, pair, 0)
        plsc.subcore_barrier()

        off = sid * rows_per_tile
        pltpu.sync_copy(deg_sh.at[pl.ds(off, rows_per_tile)],
                        deg_o.at[cid, pl.ds(off, rows_per_tile)])

    return run(dst_e, ones_k, zeros8)[0]


def _sc_pool(rows, batch, ones_k, zeros8, k, nc, ns):
    npool, hd = rows.shape
    gp = zeros8.shape[0]
    mesh = plsc.VectorSubcoreMesh(core_axis_name="c", subcore_axis_name="s")

    @functools.partial(
        pl.kernel,
        out_type=[jax.ShapeDtypeStruct((nc, gp, hd), jnp.float32),
                  jax.ShapeDtypeStruct((nc, gp, hd), jnp.float32)],
        mesh=mesh,
        scratch_types=[
            pltpu.VMEM((k,), jnp.int32),
            pltpu.VMEM((k, hd), jnp.float32),
            pltpu.VMEM((k, hd), jnp.float32),
            pltpu.VMEM_SHARED((gp, hd), jnp.float32),
            pltpu.VMEM_SHARED((gp, hd), jnp.float32),
        ],
        compiler_params=pltpu.CompilerParams(use_tc_tiling_on_sc=False))
    def run(rows_h, batch_h, ones_h, z8_h, sums_o, cnt_o,
            dst_v, rows_v, ones_v, sums_sh, cnt_sh):
        cid = lax.axis_index("c")
        sid = lax.axis_index("s")
        wid = sid * nc + cid

        @pl.when(sid == 0)
        def _zero():
            pltpu.sync_copy(z8_h, sums_sh)
            pltpu.sync_copy(z8_h, cnt_sh)

        pltpu.sync_copy(ones_h, ones_v)
        plsc.subcore_barrier()

        base = wid * k
        pltpu.sync_copy(batch_h.at[pl.ds(base, k)], dst_v)
        pltpu.sync_copy(rows_h.at[pl.ds(base, k)], rows_v)
        pltpu.sync_copy(rows_v, sums_sh.at[dst_v], add=True)
        pltpu.sync_copy(ones_v, cnt_sh.at[dst_v], add=True)
        plsc.subcore_barrier()

        @pl.when(sid == 0)
        def _out():
            pltpu.sync_copy(sums_sh, sums_o.at[cid])
            pltpu.sync_copy(cnt_sh, cnt_o.at[cid])

    return run(rows, batch, ones_k, zeros8)


def kernel(x, x_ex, DFS, STATUS, edge_index, batch,
           W1, b1, Wi1, bi1, Wh1, bh1, Wi2, bi2, Wh2, bh2,
           W2, b2, Wfc2, bfc2, Wfc3, bfc3):
    n, f_in = x.shape
    e = edge_index.shape[1]
    g = DFS.shape[0]
    hd = W1.shape[1]

    info = plsc.get_sparse_core_info()
    nc, ns = info.num_cores, info.num_subcores
    nw = nc * ns

    k = None
    if e % (4 * nw) == 0:
        k = _pick_block(e // (4 * nw), 2048, 16)
    if k is None:
        k = 2048
        epad = _round_up(e, 4 * nw * k)
    else:
        epad = e
    nchunks = epad // (nw * k)
    pad_rows = 128 if epad > e else 0
    npad = _round_up(n + pad_rows, _round_up(nw * 16, 128))
    np16 = npad * hd // 128

    src_e, dst_e = edge_index[0], edge_index[1]
    if epad > e:
        npe = epad - e
        pad_idx = (n + (jnp.arange(npe, dtype=jnp.int32) % pad_rows))
        src_e = jnp.concatenate([src_e, pad_idx])
        dst_e = jnp.concatenate([dst_e, pad_idx])

    ones_k = jnp.ones((k, hd), jnp.float32)
    zeros8 = jnp.zeros((npad, hd), jnp.float32)

    kp = npad // nw
    npool = npad
    gp = _round_up(g + 16, 16)
    batch_p = jnp.concatenate(
        [batch, jnp.full((npool - n,), g, jnp.int32)])
    ones_kp = jnp.ones((kp, hd), jnp.float32)
    zeros_g8 = jnp.zeros((gp, hd), jnp.float32)

    w1e, b1e = _expand(W1), _expand_bias(b1)
    wir1, wiz1, win1 = (_expand(Wi1[:, 0:hd]), _expand(Wi1[:, hd:2 * hd]),
                        _expand(Wi1[:, 2 * hd:3 * hd]))
    whr1, whz1, whn1 = (_expand(Wh1[:, 0:hd]), _expand(Wh1[:, hd:2 * hd]),
                        _expand(Wh1[:, 2 * hd:3 * hd]))
    bi1e = jnp.concatenate([_expand_bias(bi1[0:hd]),
                            _expand_bias(bi1[hd:2 * hd]),
                            _expand_bias(bi1[2 * hd:3 * hd])], axis=1)
    bh1e = jnp.concatenate([_expand_bias(bh1[0:hd]),
                            _expand_bias(bh1[hd:2 * hd]),
                            _expand_bias(bh1[2 * hd:3 * hd])], axis=1)
    wir2, wiz2, win2 = (_expand(Wi2[:, 0:hd]), _expand(Wi2[:, hd:2 * hd]),
                        _expand(Wi2[:, 2 * hd:3 * hd]))
    whr2, whz2, whn2 = (_expand(Wh2[:, 0:hd]), _expand(Wh2[:, hd:2 * hd]),
                        _expand(Wh2[:, 2 * hd:3 * hd]))
    bi2e = jnp.concatenate([_expand_bias(bi2[0:hd]),
                            _expand_bias(bi2[hd:2 * hd]),
                            _expand_bias(bi2[2 * hd:3 * hd])], axis=1)
    bh2e = jnp.concatenate([_expand_bias(bh2[0:hd]),
                            _expand_bias(bh2[hd:2 * hd]),
                            _expand_bias(bh2[2 * hd:3 * hd])], axis=1)
    w2e, b2e = _expand(W2), _expand_bias(b2)

    xp = jnp.concatenate([x, jnp.zeros((npad - n, f_in), jnp.float32)])
    xk = xp.reshape(np16, 128)

    x1k = _tc_lin_selu(xk, w1e, b1e)
    table1 = x1k.reshape(npad, hd)

    degp = _sc_deg(dst_e, ones_k, zeros8, k, nchunks, nc, ns, npad)
    degk = degp.reshape(nc, np16, 128)

    agg1p = _sc_edge_pass(table1, src_e, dst_e, zeros8, k, nchunks, nc, ns)
    agg1k = agg1p.reshape(nc, np16, 128)

    x2k = _tc_gru(agg1k, degk, x1k, wir1, wiz1, win1, bi1e,
                  whr1, whz1, whn1, bh1e, w2e, b2e)
    table2 = x2k.reshape(npad, hd)

    agg2p = _sc_edge_pass(table2, src_e, dst_e, zeros8, k, nchunks, nc, ns)
    agg2k = agg2p.reshape(nc, np16, 128)

    x4k = _tc_gru(agg2k, degk, x2k, wir2, wiz2, win2, bi2e,
                  whr2, whz2, whn2, bh2e, w2e, b2e)
    x4s = x4k.reshape(npad, hd)

    sumsp, cntp = _sc_pool(x4s, batch_p, ones_kp, zeros_g8, kp, nc, ns)

    x_out, x_feat = _tc_head(sumsp[:, :g], cntp[:, :g, 0:1], Wfc2, bfc2,
                             Wfc3, bfc3)
    return (x_out, x_feat)


# speedup vs baseline: 1.9808x; 1.0023x over previous
import functools

import jax
import jax.numpy as jnp
from jax import lax
from jax.experimental import pallas as pl
from jax.experimental.pallas import tpu as pltpu
from jax.experimental.pallas import tpu_sc as plsc

_SELU_ALPHA = 1.6732632423543772
_SELU_SCALE = 1.0507009873554805
_PACK = 16


def _selu(v):
    return _SELU_SCALE * jnp.where(v > 0, v, _SELU_ALPHA * (jnp.exp(v) - 1.0))


def _sigmoid(v):
    return 1.0 / (1.0 + jnp.exp(-v))


def _round_up(a, b):
    return -(-a // b) * b


def _pick_block(total, cap, mult):
    for k in range(cap - cap % mult, mult - 1, -mult):
        if total % k == 0:
            return k
    return None


def _expand(w):
    return jnp.kron(jnp.eye(_PACK, dtype=jnp.float32), w)


def _expand_bias(b):
    return jnp.tile(b, _PACK).reshape(1, -1)


def _lin_selu_body(x_ref, w_ref, b_ref, o_ref):
    o_ref[...] = _selu(
        jnp.dot(x_ref[...], w_ref[...], preferred_element_type=jnp.float32)
        + b_ref[...])


def _tc_lin_selu(xk, w_e, b_e):
    return pl.pallas_call(
        _lin_selu_body,
        out_shape=jax.ShapeDtypeStruct(xk.shape, jnp.float32),
    )(xk, w_e, b_e)


def _gru_body(aggp_ref, degp_ref, h_ref,
              wir_ref, wiz_ref, win_ref, bi_ref,
              whr_ref, whz_ref, whn_ref, bh_ref,
              w2_ref, b2_ref, o_ref):
    deg = degp_ref[0] + degp_ref[1]
    rdeg = 1.0 / jnp.maximum(deg, 1.0)
    agg = (aggp_ref[0] + aggp_ref[1]) * rdeg
    h = h_ref[...]

    def mm(a, wref):
        return jnp.dot(a, wref[...], preferred_element_type=jnp.float32)

    bi = bi_ref[...]
    bh = bh_ref[...]
    r = _sigmoid(mm(agg, wir_ref) + bi[:, 0:128]
                 + mm(h, whr_ref) + bh[:, 0:128])
    z = _sigmoid(mm(agg, wiz_ref) + bi[:, 128:256]
                 + mm(h, whz_ref) + bh[:, 128:256])
    nn = jnp.tanh(mm(agg, win_ref) + bi[:, 256:384]
                  + r * (mm(h, whn_ref) + bh[:, 256:384]))
    xn = (1.0 - z) * nn + z * h
    o_ref[...] = _selu(mm(xn, w2_ref) + b2_ref[...])


def _tc_gru(aggk, degk, hk, wir, wiz, win, bi3, whr, whz, whn, bh3, w2e, b2e):
    return pl.pallas_call(
        _gru_body,
        out_shape=jax.ShapeDtypeStruct(hk.shape, jnp.float32),
    )(aggk, degk, hk, wir, wiz, win, bi3, whr, whz, whn, bh3, w2e, b2e)


def _head_body(sums_ref, cnt_ref, w2_ref, b2_ref, w3_ref, b3_ref,
               xo_ref, xf_ref):
    sums = sums_ref[0] + sums_ref[1]
    cnt = cnt_ref[0] + cnt_ref[1]
    x5 = sums / jnp.maximum(cnt, 1.0)
    nrm = jnp.sqrt(jnp.sum(x5 * x5, axis=-1, keepdims=True))
    x7 = x5 / jnp.maximum(nrm, 1e-12)
    xf = _selu(jnp.dot(x7, w2_ref[...], preferred_element_type=jnp.float32)
               + b2_ref[...])
    xf_ref[...] = xf
    xo_ref[...] = (jnp.dot(xf, w3_ref[...], preferred_element_type=jnp.float32)
                   + b3_ref[...])


def _tc_head(sumsp, cntp, wfc2, bfc2, wfc3, bfc3):
    nc, g, hd = sumsp.shape
    f2 = wfc2.shape[1]
    return pl.pallas_call(
        _head_body,
        out_shape=(jax.ShapeDtypeStruct((g, 1), jnp.float32),
                   jax.ShapeDtypeStruct((g, f2), jnp.float32)),
    )(sumsp, cntp, wfc2, bfc2.reshape(1, f2),
      wfc3, bfc3.reshape(1, 1))


def _sc_edge_pass(table, src_e, dst_e, zeros8, k, nchunks, nc, ns):
    npad, hd = table.shape
    per_w = k * nchunks
    rows_per_tile = npad // ns
    nquads = nchunks // 4
    mesh = plsc.VectorSubcoreMesh(core_axis_name="c", subcore_axis_name="s")

    @functools.partial(
        pl.kernel,
        out_type=[jax.ShapeDtypeStruct((nc, npad, hd), jnp.float32)],
        mesh=mesh,
        scratch_types=[
            [pltpu.VMEM((k,), jnp.int32)] * 4,
            [pltpu.VMEM((k,), jnp.int32)] * 4,
            [pltpu.VMEM((k, hd), jnp.float32)] * 4,
            pltpu.VMEM_SHARED((npad, hd), jnp.float32),
            [pltpu.SemaphoreType.DMA] * 4,
        ],
        compiler_params=pltpu.CompilerParams(use_tc_tiling_on_sc=False))
    def run(table_h, srce_h, dste_h, z8_h, agg_o,
            srcs, dsts, rowss, agg_sh, gsems):
        cid = lax.axis_index("c")
        sid = lax.axis_index("s")
        wid = sid * nc + cid

        @pl.when(sid == 0)
        def _zero():
            pltpu.sync_copy(z8_h, agg_sh)
        plsc.subcore_barrier()

        base_w = wid * per_w

        for b in range(4):
            pltpu.sync_copy(srce_h.at[pl.ds(base_w + b * k, k)], srcs[b])
            pltpu.sync_copy(dste_h.at[pl.ds(base_w + b * k, k)], dsts[b])
            pltpu.async_copy(table_h.at[srcs[b]], rowss[b], gsems[b])

        def quad(j, carry):
            for b in range(4):
                pltpu.make_async_copy(table_h.at[srcs[b]], rowss[b],
                                      gsems[b]).wait()
                pltpu.sync_copy(rowss[b], agg_sh.at[dsts[b]], add=True)

                @pl.when(j < nquads - 1)
                def _pref(b=b):
                    nb = base_w + (4 * j + b + 4) * k
                    pltpu.sync_copy(srce_h.at[pl.ds(nb, k)], srcs[b])
                    pltpu.sync_copy(dste_h.at[pl.ds(nb, k)], dsts[b])
                    pltpu.async_copy(table_h.at[srcs[b]], rowss[b], gsems[b])
            return carry

        lax.fori_loop(0, nquads, quad, 0)
        plsc.subcore_barrier()

        off = sid * rows_per_tile
        pltpu.sync_copy(agg_sh.at[pl.ds(off, rows_per_tile)],
                        agg_o.at[cid, pl.ds(off, rows_per_tile)])

    return run(table, src_e, dst_e, zeros8)[0]


def _sc_deg(dst_e, ones_k, zeros8, k, nchunks, nc, ns, npad):
    hd = ones_k.shape[1]
    per_w = k * nchunks
    rows_per_tile = npad // ns
    mesh = plsc.VectorSubcoreMesh(core_axis_name="c", subcore_axis_name="s")

    @functools.partial(
        pl.kernel,
        out_type=[jax.ShapeDtypeStruct((nc, npad, hd), jnp.float32)],
        mesh=mesh,
        scratch_types=[
            pltpu.VMEM((k,), jnp.int32), pltpu.VMEM((k,), jnp.int32),
            pltpu.VMEM((k, hd), jnp.float32),
            pltpu.VMEM_SHARED((npad, hd), jnp.float32),
            pltpu.SemaphoreType.DMA,
        ],
        compiler_params=pltpu.CompilerParams(use_tc_tiling_on_sc=False))
    def run(dste_h, ones_h, z8_h, deg_o, dst0, dst1, ones_v, deg_sh, isem):
        cid = lax.axis_index("c")
        sid = lax.axis_index("s")
        wid = sid * nc + cid

        @pl.when(sid == 0)
        def _zero():
            pltpu.sync_copy(z8_h, deg_sh)
        pltpu.sync_copy(ones_h, ones_v)
        plsc.subcore_barrier()

        base_w = wid * per_w
        npairs = nchunks // 2
        pltpu.sync_copy(dste_h.at[pl.ds(base_w, k)], dst0)

        def pair(j, carry):
            b1 = base_w + (2 * j + 1) * k
            pltpu.async_copy(dste_h.at[pl.ds(b1, k)], dst1, isem)
            pltpu.sync_copy(ones_v, deg_sh.at[dst0], add=True)
            pltpu.make_async_copy(dste_h.at[pl.ds(b1, k)], dst1, isem).wait()

            @pl.when(j < npairs - 1)
            def _pref():
                b2 = base_w + (2 * j + 2) * k
                pltpu.async_copy(dste_h.at[pl.ds(b2, k)], dst0, isem)

            pltpu.sync_copy(ones_v, deg_sh.at[dst1], add=True)

            @pl.when(j < npairs - 1)
            def _drain():
                b2 = base_w + (2 * j + 2) * k
                pltpu.make_async_copy(dste_h.at[pl.ds(b2, k)], dst0,
                                      isem).wait()
            return carry

        lax.fori_loop(0, npairs, pair, 0)
        plsc.subcore_barrier()

        off = sid * rows_per_tile
        pltpu.sync_copy(deg_sh.at[pl.ds(off, rows_per_tile)],
                        deg_o.at[cid, pl.ds(off, rows_per_tile)])

    return run(dst_e, ones_k, zeros8)[0]


def _sc_pool(rows, batch, ones_k, zeros8, k, nc, ns):
    npool, hd = rows.shape
    gp = zeros8.shape[0]
    mesh = plsc.VectorSubcoreMesh(core_axis_name="c", subcore_axis_name="s")

    @functools.partial(
        pl.kernel,
        out_type=[jax.ShapeDtypeStruct((nc, gp, hd), jnp.float32),
                  jax.ShapeDtypeStruct((nc, gp, hd), jnp.float32)],
        mesh=mesh,
        scratch_types=[
            pltpu.VMEM((k,), jnp.int32),
            pltpu.VMEM((k, hd), jnp.float32),
            pltpu.VMEM((k, hd), jnp.float32),
            pltpu.VMEM_SHARED((gp, hd), jnp.float32),
            pltpu.VMEM_SHARED((gp, hd), jnp.float32),
        ],
        compiler_params=pltpu.CompilerParams(use_tc_tiling_on_sc=False))
    def run(rows_h, batch_h, ones_h, z8_h, sums_o, cnt_o,
            dst_v, rows_v, ones_v, sums_sh, cnt_sh):
        cid = lax.axis_index("c")
        sid = lax.axis_index("s")
        wid = sid * nc + cid

        @pl.when(sid == 0)
        def _zero():
            pltpu.sync_copy(z8_h, sums_sh)
            pltpu.sync_copy(z8_h, cnt_sh)

        pltpu.sync_copy(ones_h, ones_v)
        plsc.subcore_barrier()

        base = wid * k
        pltpu.sync_copy(batch_h.at[pl.ds(base, k)], dst_v)
        pltpu.sync_copy(rows_h.at[pl.ds(base, k)], rows_v)
        pltpu.sync_copy(rows_v, sums_sh.at[dst_v], add=True)
        pltpu.sync_copy(ones_v, cnt_sh.at[dst_v], add=True)
        plsc.subcore_barrier()

        @pl.when(sid == 0)
        def _out():
            pltpu.sync_copy(sums_sh, sums_o.at[cid])
            pltpu.sync_copy(cnt_sh, cnt_o.at[cid])

    return run(rows, batch, ones_k, zeros8)


def kernel(x, x_ex, DFS, STATUS, edge_index, batch,
           W1, b1, Wi1, bi1, Wh1, bh1, Wi2, bi2, Wh2, bh2,
           W2, b2, Wfc2, bfc2, Wfc3, bfc3):
    n, f_in = x.shape
    e = edge_index.shape[1]
    g = DFS.shape[0]
    hd = W1.shape[1]

    info = plsc.get_sparse_core_info()
    nc, ns = info.num_cores, info.num_subcores
    nw = nc * ns

    k = None
    if e % (4 * nw) == 0:
        k = _pick_block(e // (4 * nw), 2048, 16)
    if k is None:
        k = 2048
        epad = _round_up(e, 4 * nw * k)
    else:
        epad = e
    nchunks = epad // (nw * k)
    pad_rows = 128 if epad > e else 0
    npad = _round_up(n + pad_rows, _round_up(nw * 16, 128))
    np16 = npad * hd // 128

    src_e, dst_e = edge_index[0], edge_index[1]
    if epad > e:
        npe = epad - e
        pad_idx = (n + (jnp.arange(npe, dtype=jnp.int32) % pad_rows))
        src_e = jnp.concatenate([src_e, pad_idx])
        dst_e = jnp.concatenate([dst_e, pad_idx])

    ones_k = jnp.ones((k, hd), jnp.float32)
    zeros8 = jnp.zeros((npad, hd), jnp.float32)

    kp = npad // nw
    npool = npad
    gp = _round_up(g + 16, 16)
    batch_p = jnp.concatenate(
        [batch, jnp.full((npool - n,), g, jnp.int32)])
    ones_kp = jnp.ones((kp, hd), jnp.float32)
    zeros_g8 = jnp.zeros((gp, hd), jnp.float32)

    w1e, b1e = _expand(W1), _expand_bias(b1)
    wir1, wiz1, win1 = (_expand(Wi1[:, 0:hd]), _expand(Wi1[:, hd:2 * hd]),
                        _expand(Wi1[:, 2 * hd:3 * hd]))
    whr1, whz1, whn1 = (_expand(Wh1[:, 0:hd]), _expand(Wh1[:, hd:2 * hd]),
                        _expand(Wh1[:, 2 * hd:3 * hd]))
    bi1e = jnp.concatenate([_expand_bias(bi1[0:hd]),
                            _expand_bias(bi1[hd:2 * hd]),
                            _expand_bias(bi1[2 * hd:3 * hd])], axis=1)
    bh1e = jnp.concatenate([_expand_bias(bh1[0:hd]),
                            _expand_bias(bh1[hd:2 * hd]),
                            _expand_bias(bh1[2 * hd:3 * hd])], axis=1)
    wir2, wiz2, win2 = (_expand(Wi2[:, 0:hd]), _expand(Wi2[:, hd:2 * hd]),
                        _expand(Wi2[:, 2 * hd:3 * hd]))
    whr2, whz2, whn2 = (_expand(Wh2[:, 0:hd]), _expand(Wh2[:, hd:2 * hd]),
                        _expand(Wh2[:, 2 * hd:3 * hd]))
    bi2e = jnp.concatenate([_expand_bias(bi2[0:hd]),
                            _expand_bias(bi2[hd:2 * hd]),
                            _expand_bias(bi2[2 * hd:3 * hd])], axis=1)
    bh2e = jnp.concatenate([_expand_bias(bh2[0:hd]),
                            _expand_bias(bh2[hd:2 * hd]),
                            _expand_bias(bh2[2 * hd:3 * hd])], axis=1)
    w2e, b2e = _expand(W2), _expand_bias(b2)

    xp = jnp.concatenate([x, jnp.zeros((npad - n, f_in), jnp.float32)])
    xk = xp.reshape(np16, 128)

    degp = _sc_deg(dst_e, ones_k, zeros8, k, nchunks, nc, ns, npad)
    degk = degp.reshape(nc, np16, 128)

    x1k = _tc_lin_selu(xk, w1e, b1e)
    table1 = x1k.reshape(npad, hd)

    agg1p = _sc_edge_pass(table1, src_e, dst_e, zeros8, k, nchunks, nc, ns)
    agg1k = agg1p.reshape(nc, np16, 128)

    x2k = _tc_gru(agg1k, degk, x1k, wir1, wiz1, win1, bi1e,
                  whr1, whz1, whn1, bh1e, w2e, b2e)
    table2 = x2k.reshape(npad, hd)

    agg2p = _sc_edge_pass(table2, src_e, dst_e, zeros8, k, nchunks, nc, ns)
    agg2k = agg2p.reshape(nc, np16, 128)

    x4k = _tc_gru(agg2k, degk, x2k, wir2, wiz2, win2, bi2e,
                  whr2, whz2, whn2, bh2e, w2e, b2e)
    x4s = x4k.reshape(npad, hd)

    sumsp, cntp = _sc_pool(x4s, batch_p, ones_kp, zeros_g8, kp, nc, ns)

    x_out, x_feat = _tc_head(sumsp[:, :g], cntp[:, :g, 0:1], Wfc2, bfc2,
                             Wfc3, bfc3)
    return (x_out, x_feat)
